# Initial kernel scaffold; baseline (speedup 1.0000x reference)
import jax
import jax.numpy as jnp
from jax.experimental import pallas as pl


def kernel(node_feats, edge_feats, edge_index, W_trip, b_trip, W_score, b_score, W_self, b_self):
    raise NotImplementedError("write your pallas kernel here")


# speedup vs baseline: 1.0019x; 1.0019x over previous
import functools
import jax
import jax.numpy as jnp
from jax.experimental import pallas as pl


def _edge_block(E):
    for b in (2048, 2000, 1600, 1024, 1000, 800, 512):
        if E % b == 0:
            return b
    return 8


def _mm_edges_body(ef_ref, WeT_ref, bt_ref, WsT_ref, bs_ref, pe_ref, se_ref):
    pe = jnp.dot(ef_ref[...], WeT_ref[...], preferred_element_type=jnp.float32)
    pe = pe + bt_ref[0:1, :]
    pe_ref[...] = pe
    se = jnp.dot(pe, WsT_ref[...], preferred_element_type=jnp.float32)
    se_ref[...] = se + bs_ref[0:1, :]


def _mm_nodes_body(nf_ref, WsT_ref, WdT_ref, WselfT_ref, bself_ref,
                   WscT_ref, ps_ref, pd_ref, self_ref, ss_ref, sd_ref):
    nf = nf_ref[...]
    ps = jnp.dot(nf, WsT_ref[...], preferred_element_type=jnp.float32)
    pd = jnp.dot(nf, WdT_ref[...], preferred_element_type=jnp.float32)
    ps_ref[...] = ps
    pd_ref[...] = pd
    self_ref[...] = jnp.dot(nf, WselfT_ref[...],
                            preferred_element_type=jnp.float32) + bself_ref[0:1, :]
    ss_ref[...] = jnp.dot(ps, WscT_ref[...], preferred_element_type=jnp.float32)
    sd_ref[...] = jnp.dot(pd, WscT_ref[...], preferred_element_type=jnp.float32)


def kernel(node_feats, edge_feats, edge_index, W_trip, b_trip, W_score,
           b_score, W_self, b_self):
    N, D = node_feats.shape
    E = edge_feats.shape[0]
    NH = W_score.shape[0]
    src = edge_index[0]
    dst = edge_index[1]

    W_eT = W_trip[:, :D].T
    W_sT = W_trip[:, D:2 * D].T
    W_dT = W_trip[:, 2 * D:].T
    W_scT = W_score.T
    W_selfT = W_self.T
    bt = jnp.broadcast_to(b_trip[None, :], (8, D))
    bsc = jnp.broadcast_to(b_score[None, :], (8, NH))
    bself = jnp.broadcast_to(b_self[None, :], (8, D))

    BE = _edge_block(E)
    pe, se = pl.pallas_call(
        _mm_edges_body,
        grid=(E // BE,),
        in_specs=[
            pl.BlockSpec((BE, D), lambda i: (i, 0)),
            pl.BlockSpec((D, D), lambda i: (0, 0)),
            pl.BlockSpec((8, D), lambda i: (0, 0)),
            pl.BlockSpec((D, NH), lambda i: (0, 0)),
            pl.BlockSpec((8, NH), lambda i: (0, 0)),
        ],
        out_specs=[
            pl.BlockSpec((BE, D), lambda i: (i, 0)),
            pl.BlockSpec((BE, NH), lambda i: (i, 0)),
        ],
        out_shape=[
            jax.ShapeDtypeStruct((E, D), jnp.float32),
            jax.ShapeDtypeStruct((E, NH), jnp.float32),
        ],
    )(edge_feats, W_eT, bt, W_scT, bsc)

    BN = _edge_block(N)
    ps, pd, self_msg, ss, sd = pl.pallas_call(
        _mm_nodes_body,
        grid=(N // BN,),
        in_specs=[
            pl.BlockSpec((BN, D), lambda i: (i, 0)),
            pl.BlockSpec((D, D), lambda i: (0, 0)),
            pl.BlockSpec((D, D), lambda i: (0, 0)),
            pl.BlockSpec((D, D), lambda i: (0, 0)),
            pl.BlockSpec((8, D), lambda i: (0, 0)),
            pl.BlockSpec((D, NH), lambda i: (0, 0)),
        ],
        out_specs=[
            pl.BlockSpec((BN, D), lambda i: (i, 0)),
            pl.BlockSpec((BN, D), lambda i: (i, 0)),
            pl.BlockSpec((BN, D), lambda i: (i, 0)),
            pl.BlockSpec((BN, NH), lambda i: (i, 0)),
            pl.BlockSpec((BN, NH), lambda i: (i, 0)),
        ],
        out_shape=[
            jax.ShapeDtypeStruct((N, D), jnp.float32),
            jax.ShapeDtypeStruct((N, D), jnp.float32),
            jax.ShapeDtypeStruct((N, D), jnp.float32),
            jax.ShapeDtypeStruct((N, NH), jnp.float32),
            jax.ShapeDtypeStruct((N, NH), jnp.float32),
        ],
    )(node_feats, W_sT, W_dT, W_selfT, bself, W_scT)

    score = se + ss[src] + sd[dst]
    score = jnp.where(score >= 0, score, 0.01 * score)
    smax = jax.ops.segment_max(score, dst, num_segments=N)
    ex = jnp.exp(score - smax[dst])
    denom = jax.ops.segment_sum(ex, dst, num_segments=N)
    w = jnp.mean(ex / denom[dst], axis=1, keepdims=True)
    acc = jax.ops.segment_sum(w * (pe + ps[src]), dst, num_segments=N)
    has_edge = (denom[:, :1] > 0).astype(jnp.float32)
    out = acc + has_edge * pd + self_msg
    slope = (1.0 / 8.0 + 1.0 / 3.0) / 2.0
    return jnp.where(out >= 0, out, slope * out)
